# baseline (speedup 1.0000x reference)
import jax
import jax.numpy as jnp
from jax.experimental import pallas as pl
from jax.experimental.pallas import tpu as pltpu

_B, _L, _NV, _DV, _DT, _DH = 128, 80, 36, 12544, 300, 512
_LAM = 9.0
_BF = jnp.bfloat16

_MBLK = 2304
_KBLK = 896
_MT = (_B * _NV) // _MBLK
_KT = _DV // _KBLK


def _img_body(x_ref, w_ref, b_ref, o_ref, acc_ref):
    k = pl.program_id(1)

    @pl.when(k == 0)
    def _():
        acc_ref[...] = jnp.zeros_like(acc_ref)

    acc_ref[...] += jax.lax.dot_general(
        x_ref[...].astype(_BF), w_ref[...].astype(_BF), (((1,), (1,)), ((), ())),
        preferred_element_type=jnp.float32)

    @pl.when(k == _KT - 1)
    def _():
        o_ref[...] = acc_ref[...] + b_ref[...]


def _img_linear(detect16, img_W16, img_b2):
    return pl.pallas_call(
        _img_body,
        grid=(_MT, _KT),
        in_specs=[
            pl.BlockSpec((_MBLK, _KBLK), lambda i, k: (i, k)),
            pl.BlockSpec((_DH, _KBLK), lambda i, k: (0, k)),
            pl.BlockSpec((1, _DH), lambda i, k: (0, 0)),
        ],
        out_specs=pl.BlockSpec((_MBLK, _DH), lambda i, k: (i, 0)),
        out_shape=jax.ShapeDtypeStruct((_B * _NV, _DH), jnp.float32),
        scratch_shapes=[pltpu.VMEM((_MBLK, _DH), jnp.float32)],
        compiler_params=pltpu.CompilerParams(
            dimension_semantics=("parallel", "arbitrary"),
            vmem_limit_bytes=56 * 1024 * 1024),
        name="img_linear",
    )(detect16, img_W16, img_b2)


_BB = 8
_GB = _B // _BB


def _shift(x, d):
    if d > 0:
        return jnp.concatenate([x[:, d:, :]] + [x[:, _L - 1:, :]] * d, axis=1)
    if d < 0:
        return jnp.concatenate([x[:, :1, :]] * (-d) + [x[:, : _L + d, :]], axis=1)
    return x


def _bdot(a16, b16):
    return jax.lax.dot_general(a16, b16, (((1,), (0,)), ((), ())),
                               preferred_element_type=jnp.float32)


def _bdot_nt(a16, b16):
    return jax.lax.dot_general(a16, b16, (((1,), (1,)), ((), ())),
                               preferred_element_type=jnp.float32)


def _bdot_tn(a16, b16):
    return jax.lax.dot_general(a16, b16, (((0,), (0,)), ((), ())),
                               preferred_element_type=jnp.float32)


def _txt_body(x_ref, w1_ref, w2_ref, w3_ref, w7_ref,
              b1_ref, b2_ref, b3_ref, b7_ref, txtB_ref, tb_ref, t_out):
    x = x_ref[...]

    def sh(d):
        return _shift(x, d).reshape(_BB * _L, _DT).astype(_BF)

    sm1, s0, s1 = sh(-1), sh(0), sh(1)
    x1 = jnp.tanh(_bdot(s0, w1_ref[0]) + b1_ref[...]).astype(_BF)
    acc = _bdot(x1, txtB_ref[0]) + tb_ref[...]
    s01 = jnp.concatenate([s0, s1], axis=1)
    x2 = jnp.tanh(_bdot(s01, w2_ref[...].reshape(2 * _DT, _DT))
                  + b2_ref[...]).astype(_BF)
    acc = acc + _bdot(x2, txtB_ref[1])
    sm11 = jnp.concatenate([sm1, s01], axis=1)
    x3 = jnp.tanh(_bdot(sm11, w3_ref[...].reshape(3 * _DT, _DT))
                  + b3_ref[...]).astype(_BF)
    acc = acc + _bdot(x3, txtB_ref[2]) + _bdot(x3, txtB_ref[3])
    s7 = jnp.concatenate([sh(-3), sh(-2), sm11, sh(2), sh(3)], axis=1)
    x7 = jnp.tanh(_bdot(s7, w7_ref[...].reshape(7 * _DT, _DT))
                  + b7_ref[...]).astype(_BF)
    acc = acc + _bdot(x7, txtB_ref[4])
    t_out[...] = jnp.tanh(acc)


def _txt_call(txts, w1, w2, w3, w7, b1, b2, b3, b7, txtB, tb):
    full = lambda shape: pl.BlockSpec(shape, lambda c, j: tuple(0 for _ in shape))
    return pl.pallas_call(
        _txt_body,
        grid=(2, _GB // 2),
        in_specs=[
            pl.BlockSpec((_BB, _L, _DT), lambda c, j: (c * (_GB // 2) + j, 0, 0)),
            full((1, _DT, _DT)), full((2, _DT, _DT)), full((3, _DT, _DT)),
            full((7, _DT, _DT)),
            full((1, _DT)), full((1, _DT)), full((1, _DT)), full((1, _DT)),
            full((5, _DT, _DH)), full((1, _DH)),
        ],
        out_specs=pl.BlockSpec((_BB * _L, _DH),
                               lambda c, j: (c * (_GB // 2) + j, 0)),
        out_shape=jax.ShapeDtypeStruct((_B * _L, _DH), jnp.float32),
        compiler_params=pltpu.CompilerParams(
            dimension_semantics=("parallel", "arbitrary"),
            vmem_limit_bytes=56 * 1024 * 1024),
        name="txtnet",
    )(txts, w1, w2, w3, w7, b1, b2, b3, b7, txtB, tb)


def _att_body(t_ref, v_ref, tW_ref, t_b_ref, vW_ref, v_b_ref,
              ctw_ref, ctb_ref, cvw_ref, cvb_ref,
              vat_out, tav_out, attt_out, attv_out):
    tt = t_ref[...]
    tt16 = tt.astype(_BF)
    ct2 = jnp.tanh(_bdot(tt16, tW_ref[...]) + t_b_ref[...]).astype(_BF)

    for i in range(_BB):
        t_i = tt[i * _L:(i + 1) * _L]
        t16 = tt16[i * _L:(i + 1) * _L]
        v_i = v_ref[i]
        v16 = v_i.astype(_BF)

        tn = jnp.sqrt(jnp.sum(t_i * t_i, axis=1, keepdims=True))
        vn1 = jnp.sqrt(jnp.sum(v_i * v_i, axis=1))
        sim = _bdot_nt(t16, v16)
        sim_n = sim / jnp.maximum(tn * vn1[None, :], 1e-8)

        a = jnp.maximum(sim_n, 0.0)
        ar = a / jnp.maximum(jnp.sum(a, axis=1, keepdims=True), 1e-10)
        er = jnp.exp(ar * _LAM)
        w_v2t = er / jnp.sum(er, axis=1, keepdims=True)
        vat_out[i] = _bdot_tn(w_v2t.astype(_BF), t16)

        ac = a / jnp.maximum(jnp.sum(a, axis=0, keepdims=True), 1e-10)
        ec = jnp.exp(ac * _LAM)
        w_t2v_t = ec / jnp.sum(ec, axis=0, keepdims=True)
        tav_out[i * _L:(i + 1) * _L] = _bdot(w_t2v_t.astype(_BF), v16)

        c_t = jnp.mean(t_i, axis=0, keepdims=True)
        ct1 = jnp.tanh(ctw_ref[...] * c_t + ctb_ref[...]).astype(_BF)
        lg_t = _bdot_nt(ct1, ct2[i * _L:(i + 1) * _L]) * _LAM
        mt = jnp.max(lg_t, axis=1, keepdims=True)
        et = jnp.exp(lg_t - mt)
        wt = (et / jnp.sum(et, axis=1, keepdims=True)).astype(_BF)
        attt_out[i] = _bdot(wt, t16)

        cv2 = jnp.tanh(_bdot(v16, vW_ref[...]) + v_b_ref[...]).astype(_BF)
        c_v = jnp.mean(v_i, axis=0, keepdims=True)
        cv1 = jnp.tanh(cvw_ref[...] * c_v + cvb_ref[...]).astype(_BF)
        lg_v = _bdot_nt(cv1, cv2) * _LAM
        mv = jnp.max(lg_v, axis=1, keepdims=True)
        ev = jnp.exp(lg_v - mv)
        wv = (ev / jnp.sum(ev, axis=1, keepdims=True)).astype(_BF)
        attv_out[i] = _bdot(wv, v16)


def _att_call(t2d, v3, tW16, t_b, vW16, v_b, ctw, ctb, cvw, cvb):
    full = lambda shape: pl.BlockSpec(shape, lambda c, j: tuple(0 for _ in shape))
    out_shapes = (
        jax.ShapeDtypeStruct((_B, _NV, _DH), jnp.float32),
        jax.ShapeDtypeStruct((_B * _L, _DH), jnp.float32),
        jax.ShapeDtypeStruct((_B, 6, _DH), jnp.float32),
        jax.ShapeDtypeStruct((_B, 6, _DH), jnp.float32),
    )
    _ix3 = lambda c, j: (c * (_GB // 2) + j, 0, 0)
    _ix2 = lambda c, j: (c * (_GB // 2) + j, 0)
    out_specs = (
        pl.BlockSpec((_BB, _NV, _DH), _ix3),
        pl.BlockSpec((_BB * _L, _DH), _ix2),
        pl.BlockSpec((_BB, 6, _DH), _ix3),
        pl.BlockSpec((_BB, 6, _DH), _ix3),
    )
    return pl.pallas_call(
        _att_body,
        grid=(2, _GB // 2),
        in_specs=[
            pl.BlockSpec((_BB * _L, _DH), lambda c, j: (c * (_GB // 2) + j, 0)),
            pl.BlockSpec((_BB, _NV, _DH),
                         lambda c, j: (c * (_GB // 2) + j, 0, 0)),
            full((_DH, _DH)), full((1, _DH)),
            full((_DH, _DH)), full((1, _DH)),
            full((6, _DH)), full((6, _DH)), full((6, _DH)), full((6, _DH)),
        ],
        out_specs=out_specs,
        out_shape=out_shapes,
        compiler_params=pltpu.CompilerParams(
            dimension_semantics=("parallel", "arbitrary"),
            vmem_limit_bytes=56 * 1024 * 1024),
        name="oan_attention",
    )(t2d, v3, tW16, t_b, vW16, v_b, ctw, ctb, cvw, cvb)


def kernel(txts, detect, img_W, img_b, conv1_W, conv1_b, conv2_W, conv2_b,
           conv3_W, conv3_b, conv7_W, conv7_b, txt_W, txt_b,
           ct_W, ct_b, t_W, t_b, cv_W, cv_b, v_W, v_b):
    detect2 = detect.reshape(_B * _NV, _DV)

    w1 = conv1_W.astype(_BF).transpose(2, 1, 0)
    w2 = conv2_W.astype(_BF).transpose(2, 1, 0)
    w3 = conv3_W.astype(_BF).transpose(2, 1, 0)
    w7 = conv7_W.astype(_BF).transpose(2, 1, 0)
    txt_W16 = txt_W.astype(_BF)
    txtB = jnp.stack([txt_W16[:, j * _DT:(j + 1) * _DT].T for j in range(5)])

    t2d = _txt_call(txts, w1, w2, w3, w7,
                    conv1_b.reshape(1, _DT), conv2_b.reshape(1, _DT),
                    conv3_b.reshape(1, _DT), conv7_b.reshape(1, _DT),
                    txtB, txt_b.reshape(1, _DH))

    v_flat = _img_linear(detect2, img_W, img_b.reshape(1, _DH))
    v3 = v_flat.reshape(_B, _NV, _DH)

    ctw = jnp.tile(ct_W[:6, 0:1], (1, _DH))
    ctb = jnp.tile(ct_b[:6, None], (1, _DH))
    cvw = jnp.tile(cv_W[:6, 0:1], (1, _DH))
    cvb = jnp.tile(cv_b[:6, None], (1, _DH))

    vat, tav2d, att_text, att_vis = _att_call(
        t2d, v3, t_W.T.astype(_BF), t_b.reshape(1, _DH),
        v_W.T.astype(_BF), v_b.reshape(1, _DH), ctw, ctb, cvw, cvb)

    t = t2d.reshape(_B, _L, _DH)
    tav = tav2d.reshape(_B, _L, _DH)
    return (v3, t, vat, tav, att_text, att_vis)


# speedup vs baseline: 1.1083x; 1.1083x over previous
import jax
import jax.numpy as jnp
from jax.experimental import pallas as pl
from jax.experimental.pallas import tpu as pltpu

_B, _L, _NV, _DV, _DT, _DH = 128, 80, 36, 12544, 300, 512
_LAM = 9.0
_BF = jnp.bfloat16

_BB1 = 16
_KBLK = 1792
_KT = _DV // _KBLK


def _img_body(x_ref, w_ref, b_ref, o_ref, acc_ref):
    k = pl.program_id(1)

    @pl.when(k == 0)
    def _():
        acc_ref[...] = jnp.zeros_like(acc_ref)

    xb = x_ref[...]
    merged = jnp.concatenate([xb[b] for b in range(_BB1)], axis=0)
    acc_ref[...] += jax.lax.dot_general(
        merged.astype(_BF), w_ref[...].astype(_BF), (((1,), (1,)), ((), ())),
        preferred_element_type=jnp.float32)

    @pl.when(k == _KT - 1)
    def _():
        o_ref[...] = acc_ref[...] + b_ref[...]


def _img_linear(detect, img_W, img_b2):
    return pl.pallas_call(
        _img_body,
        grid=(_B // _BB1, _KT),
        in_specs=[
            pl.BlockSpec((_BB1, _NV, _KBLK), lambda i, k: (i, 0, k)),
            pl.BlockSpec((_DH, _KBLK), lambda i, k: (0, k)),
            pl.BlockSpec((1, _DH), lambda i, k: (0, 0)),
        ],
        out_specs=pl.BlockSpec((_BB1 * _NV, _DH), lambda i, k: (i, 0)),
        out_shape=jax.ShapeDtypeStruct((_B * _NV, _DH), jnp.float32),
        scratch_shapes=[pltpu.VMEM((_BB1 * _NV, _DH), jnp.float32)],
        compiler_params=pltpu.CompilerParams(
            dimension_semantics=("parallel", "arbitrary"),
            vmem_limit_bytes=56 * 1024 * 1024),
        name="img_linear",
    )(detect, img_W, img_b2)


_BB = 8
_GB = _B // _BB


def _shift(x, d):
    if d > 0:
        return jnp.concatenate([x[:, d:, :]] + [x[:, _L - 1:, :]] * d, axis=1)
    if d < 0:
        return jnp.concatenate([x[:, :1, :]] * (-d) + [x[:, : _L + d, :]], axis=1)
    return x


def _bdot(a16, b16):
    return jax.lax.dot_general(a16, b16, (((1,), (0,)), ((), ())),
                               preferred_element_type=jnp.float32)


def _bdot_nt(a16, b16):
    return jax.lax.dot_general(a16, b16, (((1,), (1,)), ((), ())),
                               preferred_element_type=jnp.float32)


def _bdot_tn(a16, b16):
    return jax.lax.dot_general(a16, b16, (((0,), (0,)), ((), ())),
                               preferred_element_type=jnp.float32)


def _txt_body(x_ref, w1_ref, w2_ref, w3_ref, w7_ref,
              b1_ref, b2_ref, b3_ref, b7_ref, txtB_ref, tb_ref, t_out):
    x = x_ref[...]

    def sh(d):
        return _shift(x, d).reshape(_BB * _L, _DT).astype(_BF)

    sm1, s0, s1 = sh(-1), sh(0), sh(1)
    x1 = jnp.tanh(_bdot(s0, w1_ref[0]) + b1_ref[...]).astype(_BF)
    acc = _bdot(x1, txtB_ref[0]) + tb_ref[...]
    s01 = jnp.concatenate([s0, s1], axis=1)
    x2 = jnp.tanh(_bdot(s01, w2_ref[...].reshape(2 * _DT, _DT))
                  + b2_ref[...]).astype(_BF)
    acc = acc + _bdot(x2, txtB_ref[1])
    sm11 = jnp.concatenate([sm1, s01], axis=1)
    x3 = jnp.tanh(_bdot(sm11, w3_ref[...].reshape(3 * _DT, _DT))
                  + b3_ref[...]).astype(_BF)
    acc = acc + _bdot(x3, txtB_ref[2]) + _bdot(x3, txtB_ref[3])
    s7 = jnp.concatenate([sh(-3), sh(-2), sm11, sh(2), sh(3)], axis=1)
    x7 = jnp.tanh(_bdot(s7, w7_ref[...].reshape(7 * _DT, _DT))
                  + b7_ref[...]).astype(_BF)
    acc = acc + _bdot(x7, txtB_ref[4])
    t_out[...] = jnp.tanh(acc)


def _txt_call(txts, w1, w2, w3, w7, b1, b2, b3, b7, txtB, tb):
    full = lambda shape: pl.BlockSpec(shape, lambda c, j: tuple(0 for _ in shape))
    return pl.pallas_call(
        _txt_body,
        grid=(2, _GB // 2),
        in_specs=[
            pl.BlockSpec((_BB, _L, _DT), lambda c, j: (c * (_GB // 2) + j, 0, 0)),
            full((1, _DT, _DT)), full((2, _DT, _DT)), full((3, _DT, _DT)),
            full((7, _DT, _DT)),
            full((1, _DT)), full((1, _DT)), full((1, _DT)), full((1, _DT)),
            full((5, _DT, _DH)), full((1, _DH)),
        ],
        out_specs=pl.BlockSpec((_BB * _L, _DH),
                               lambda c, j: (c * (_GB // 2) + j, 0)),
        out_shape=jax.ShapeDtypeStruct((_B * _L, _DH), jnp.float32),
        compiler_params=pltpu.CompilerParams(
            dimension_semantics=("parallel", "arbitrary"),
            vmem_limit_bytes=56 * 1024 * 1024),
        name="txtnet",
    )(txts, w1, w2, w3, w7, b1, b2, b3, b7, txtB, tb)


def _att_body(t_ref, v_ref, tW_ref, t_b_ref, vW_ref, v_b_ref,
              ctw_ref, ctb_ref, cvw_ref, cvb_ref,
              vat_out, tav_out, attt_out, attv_out):
    tt = t_ref[...]
    tt16 = tt.astype(_BF)
    ct2 = jnp.tanh(_bdot(tt16, tW_ref[...]) + t_b_ref[...]).astype(_BF)

    for i in range(_BB):
        t_i = tt[i * _L:(i + 1) * _L]
        t16 = tt16[i * _L:(i + 1) * _L]
        v_i = v_ref[i]
        v16 = v_i.astype(_BF)

        tn = jnp.sqrt(jnp.sum(t_i * t_i, axis=1, keepdims=True))
        vn1 = jnp.sqrt(jnp.sum(v_i * v_i, axis=1))
        sim = _bdot_nt(t16, v16)
        sim_n = sim / jnp.maximum(tn * vn1[None, :], 1e-8)

        a = jnp.maximum(sim_n, 0.0)
        ar = a / jnp.maximum(jnp.sum(a, axis=1, keepdims=True), 1e-10)
        er = jnp.exp(ar * _LAM)
        w_v2t = er / jnp.sum(er, axis=1, keepdims=True)
        vat_out[i] = _bdot_tn(w_v2t.astype(_BF), t16)

        ac = a / jnp.maximum(jnp.sum(a, axis=0, keepdims=True), 1e-10)
        ec = jnp.exp(ac * _LAM)
        w_t2v_t = ec / jnp.sum(ec, axis=0, keepdims=True)
        tav_out[i * _L:(i + 1) * _L] = _bdot(w_t2v_t.astype(_BF), v16)

        c_t = jnp.mean(t_i, axis=0, keepdims=True)
        ct1 = jnp.tanh(ctw_ref[...] * c_t + ctb_ref[...]).astype(_BF)
        lg_t = _bdot_nt(ct1, ct2[i * _L:(i + 1) * _L]) * _LAM
        mt = jnp.max(lg_t, axis=1, keepdims=True)
        et = jnp.exp(lg_t - mt)
        wt = (et / jnp.sum(et, axis=1, keepdims=True)).astype(_BF)
        attt_out[i] = _bdot(wt, t16)

        cv2 = jnp.tanh(_bdot(v16, vW_ref[...]) + v_b_ref[...]).astype(_BF)
        c_v = jnp.mean(v_i, axis=0, keepdims=True)
        cv1 = jnp.tanh(cvw_ref[...] * c_v + cvb_ref[...]).astype(_BF)
        lg_v = _bdot_nt(cv1, cv2) * _LAM
        mv = jnp.max(lg_v, axis=1, keepdims=True)
        ev = jnp.exp(lg_v - mv)
        wv = (ev / jnp.sum(ev, axis=1, keepdims=True)).astype(_BF)
        attv_out[i] = _bdot(wv, v16)


def _att_call(t2d, v3, tW16, t_b, vW16, v_b, ctw, ctb, cvw, cvb):
    full = lambda shape: pl.BlockSpec(shape, lambda c, j: tuple(0 for _ in shape))
    out_shapes = (
        jax.ShapeDtypeStruct((_B, _NV, _DH), jnp.float32),
        jax.ShapeDtypeStruct((_B * _L, _DH), jnp.float32),
        jax.ShapeDtypeStruct((_B, 6, _DH), jnp.float32),
        jax.ShapeDtypeStruct((_B, 6, _DH), jnp.float32),
    )
    _ix3 = lambda c, j: (c * (_GB // 2) + j, 0, 0)
    _ix2 = lambda c, j: (c * (_GB // 2) + j, 0)
    out_specs = (
        pl.BlockSpec((_BB, _NV, _DH), _ix3),
        pl.BlockSpec((_BB * _L, _DH), _ix2),
        pl.BlockSpec((_BB, 6, _DH), _ix3),
        pl.BlockSpec((_BB, 6, _DH), _ix3),
    )
    return pl.pallas_call(
        _att_body,
        grid=(2, _GB // 2),
        in_specs=[
            pl.BlockSpec((_BB * _L, _DH), lambda c, j: (c * (_GB // 2) + j, 0)),
            pl.BlockSpec((_BB, _NV, _DH),
                         lambda c, j: (c * (_GB // 2) + j, 0, 0)),
            full((_DH, _DH)), full((1, _DH)),
            full((_DH, _DH)), full((1, _DH)),
            full((6, _DH)), full((6, _DH)), full((6, _DH)), full((6, _DH)),
        ],
        out_specs=out_specs,
        out_shape=out_shapes,
        compiler_params=pltpu.CompilerParams(
            dimension_semantics=("parallel", "arbitrary"),
            vmem_limit_bytes=56 * 1024 * 1024),
        name="oan_attention",
    )(t2d, v3, tW16, t_b, vW16, v_b, ctw, ctb, cvw, cvb)


def kernel(txts, detect, img_W, img_b, conv1_W, conv1_b, conv2_W, conv2_b,
           conv3_W, conv3_b, conv7_W, conv7_b, txt_W, txt_b,
           ct_W, ct_b, t_W, t_b, cv_W, cv_b, v_W, v_b):
    w1 = conv1_W.astype(_BF).transpose(2, 1, 0)
    w2 = conv2_W.astype(_BF).transpose(2, 1, 0)
    w3 = conv3_W.astype(_BF).transpose(2, 1, 0)
    w7 = conv7_W.astype(_BF).transpose(2, 1, 0)
    txt_W16 = txt_W.astype(_BF)
    txtB = jnp.stack([txt_W16[:, j * _DT:(j + 1) * _DT].T for j in range(5)])

    t2d = _txt_call(txts, w1, w2, w3, w7,
                    conv1_b.reshape(1, _DT), conv2_b.reshape(1, _DT),
                    conv3_b.reshape(1, _DT), conv7_b.reshape(1, _DT),
                    txtB, txt_b.reshape(1, _DH))

    v_flat = _img_linear(detect, img_W, img_b.reshape(1, _DH))
    v3 = v_flat.reshape(_B, _NV, _DH)

    ctw = jnp.tile(ct_W[:6, 0:1], (1, _DH))
    ctb = jnp.tile(ct_b[:6, None], (1, _DH))
    cvw = jnp.tile(cv_W[:6, 0:1], (1, _DH))
    cvb = jnp.tile(cv_b[:6, None], (1, _DH))

    vat, tav2d, att_text, att_vis = _att_call(
        t2d, v3, t_W.T.astype(_BF), t_b.reshape(1, _DH),
        v_W.T.astype(_BF), v_b.reshape(1, _DH), ctw, ctb, cvw, cvb)

    t = t2d.reshape(_B, _L, _DH)
    tav = tav2d.reshape(_B, _L, _DH)
    return (v3, t, vat, tav, att_text, att_vis)


# speedup vs baseline: 1.1910x; 1.0747x over previous
import jax
import jax.numpy as jnp
from jax.experimental import pallas as pl
from jax.experimental.pallas import tpu as pltpu

_B, _L, _NV, _DV, _DT, _DH = 128, 80, 36, 12544, 300, 512
_LAM = 9.0
_BF = jnp.bfloat16

_BB1 = 16
_KBLK = 1792
_KT = _DV // _KBLK


def _img_body(x_ref, w_ref, b_ref, o_ref, acc_ref):
    k = pl.program_id(1)

    @pl.when(k == 0)
    def _():
        acc_ref[...] = jnp.zeros_like(acc_ref)

    xb = x_ref[...]
    merged = jnp.concatenate([xb[b] for b in range(_BB1)], axis=0)
    acc_ref[...] += jax.lax.dot_general(
        merged.astype(_BF), w_ref[...].astype(_BF), (((1,), (1,)), ((), ())),
        preferred_element_type=jnp.float32)

    @pl.when(k == _KT - 1)
    def _():
        o_ref[...] = acc_ref[...] + b_ref[...]


def _img_linear(detect, img_W, img_b2):
    return pl.pallas_call(
        _img_body,
        grid=(_B // _BB1, _KT),
        in_specs=[
            pl.BlockSpec((_BB1, _NV, _KBLK), lambda i, k: (i, 0, k)),
            pl.BlockSpec((_DH, _KBLK), lambda i, k: (0, k)),
            pl.BlockSpec((1, _DH), lambda i, k: (0, 0)),
        ],
        out_specs=pl.BlockSpec((_BB1 * _NV, _DH), lambda i, k: (i, 0)),
        out_shape=jax.ShapeDtypeStruct((_B * _NV, _DH), jnp.float32),
        scratch_shapes=[pltpu.VMEM((_BB1 * _NV, _DH), jnp.float32)],
        compiler_params=pltpu.CompilerParams(
            dimension_semantics=("parallel", "arbitrary"),
            vmem_limit_bytes=56 * 1024 * 1024),
        name="img_linear",
    )(detect, img_W, img_b2)


_BB = 8
_GB = _B // _BB


def _shift(x, d):
    if d > 0:
        return jnp.concatenate([x[:, d:, :]] + [x[:, _L - 1:, :]] * d, axis=1)
    if d < 0:
        return jnp.concatenate([x[:, :1, :]] * (-d) + [x[:, : _L + d, :]], axis=1)
    return x


def _bdot(a16, b16):
    return jax.lax.dot_general(a16, b16, (((1,), (0,)), ((), ())),
                               preferred_element_type=jnp.float32)


def _bdot_nt(a16, b16):
    return jax.lax.dot_general(a16, b16, (((1,), (1,)), ((), ())),
                               preferred_element_type=jnp.float32)


def _bdot_tn(a16, b16):
    return jax.lax.dot_general(a16, b16, (((0,), (0,)), ((), ())),
                               preferred_element_type=jnp.float32)


def _txt_body(x_ref, w1_ref, w2_ref, w3_ref, w7_ref,
              b1_ref, b2_ref, b3_ref, b7_ref, txtB_ref, tb_ref, t_out):
    x = x_ref[...]

    def sh(d):
        return _shift(x, d).reshape(_BB * _L, _DT).astype(_BF)

    sm1, s0, s1 = sh(-1), sh(0), sh(1)
    x1 = jnp.tanh(_bdot(s0, w1_ref[0]) + b1_ref[...]).astype(_BF)
    acc = _bdot(x1, txtB_ref[0]) + tb_ref[...]
    s01 = jnp.concatenate([s0, s1], axis=1)
    x2 = jnp.tanh(_bdot(s01, w2_ref[...].reshape(2 * _DT, _DT))
                  + b2_ref[...]).astype(_BF)
    acc = acc + _bdot(x2, txtB_ref[1])
    sm11 = jnp.concatenate([sm1, s01], axis=1)
    x3 = jnp.tanh(_bdot(sm11, w3_ref[...].reshape(3 * _DT, _DT))
                  + b3_ref[...]).astype(_BF)
    acc = acc + _bdot(x3, txtB_ref[2]) + _bdot(x3, txtB_ref[3])
    s7 = jnp.concatenate([sh(-3), sh(-2), sm11, sh(2), sh(3)], axis=1)
    x7 = jnp.tanh(_bdot(s7, w7_ref[...].reshape(7 * _DT, _DT))
                  + b7_ref[...]).astype(_BF)
    acc = acc + _bdot(x7, txtB_ref[4])
    t_out[...] = jnp.tanh(acc)


def _txt_call(txts, w1, w2, w3, w7, b1, b2, b3, b7, txtB, tb):
    full = lambda shape: pl.BlockSpec(shape, lambda c, j: tuple(0 for _ in shape))
    return pl.pallas_call(
        _txt_body,
        grid=(2, _GB // 2),
        in_specs=[
            pl.BlockSpec((_BB, _L, _DT), lambda c, j: (c * (_GB // 2) + j, 0, 0)),
            full((1, _DT, _DT)), full((2, _DT, _DT)), full((3, _DT, _DT)),
            full((7, _DT, _DT)),
            full((1, _DT)), full((1, _DT)), full((1, _DT)), full((1, _DT)),
            full((5, _DT, _DH)), full((1, _DH)),
        ],
        out_specs=pl.BlockSpec((_BB * _L, _DH),
                               lambda c, j: (c * (_GB // 2) + j, 0)),
        out_shape=jax.ShapeDtypeStruct((_B * _L, _DH), jnp.float32),
        compiler_params=pltpu.CompilerParams(
            dimension_semantics=("parallel", "arbitrary"),
            vmem_limit_bytes=56 * 1024 * 1024),
        name="txtnet",
    )(txts, w1, w2, w3, w7, b1, b2, b3, b7, txtB, tb)


def _att_body(t_ref, v_ref, tW_ref, t_b_ref, vW_ref, v_b_ref,
              ctw_ref, ctb_ref, cvw_ref, cvb_ref,
              vat_out, tav_out, attt_out, attv_out):
    tt = t_ref[...]
    tt16 = tt.astype(_BF)
    ct2 = jnp.tanh(_bdot(tt16, tW_ref[...]) + t_b_ref[...]).astype(_BF)

    tn = jnp.sqrt(jnp.sum(tt * tt, axis=1, keepdims=True))
    sims = []
    dens = []
    for i in range(_BB):
        t16 = tt16[i * _L:(i + 1) * _L]
        v_i = v_ref[i]
        v16 = v_i.astype(_BF)
        sims.append(_bdot_nt(t16, v16))
        vn1 = jnp.sqrt(jnp.sum(v_i * v_i, axis=1))
        dens.append(jnp.broadcast_to(vn1[None, :], (_L, _NV)))
    sim = jnp.concatenate(sims, axis=0)
    vnb = jnp.concatenate(dens, axis=0)
    sim_n = sim / jnp.maximum(tn * vnb, 1e-8)

    a = jnp.maximum(sim_n, 0.0)
    ar = a / jnp.maximum(jnp.sum(a, axis=1, keepdims=True), 1e-10)
    er = jnp.exp(ar * _LAM)
    w_v2t = (er / jnp.sum(er, axis=1, keepdims=True)).astype(_BF)

    a3 = a.reshape(_BB, _L, _NV)
    ac = a3 / jnp.maximum(jnp.sum(a3, axis=1, keepdims=True), 1e-10)
    ec = jnp.exp(ac * _LAM)
    w_t2v = (ec / jnp.sum(ec, axis=1, keepdims=True)).reshape(_BB * _L, _NV)
    w_t2v = w_t2v.astype(_BF)

    tsum3 = jnp.sum(tt.reshape(_BB, _L, _DH), axis=1, keepdims=True)

    for i in range(_BB):
        t16 = tt16[i * _L:(i + 1) * _L]
        v_i = v_ref[i]
        v16 = v_i.astype(_BF)
        vat_out[i] = _bdot_tn(w_v2t[i * _L:(i + 1) * _L], t16)
        tav_out[i * _L:(i + 1) * _L] = _bdot(w_t2v[i * _L:(i + 1) * _L], v16)

        c_t = tsum3[i] * (1.0 / _L)
        ct1 = jnp.tanh(ctw_ref[...] * c_t + ctb_ref[...]).astype(_BF)
        lg_t = _bdot_nt(ct1, ct2[i * _L:(i + 1) * _L]) * _LAM
        mt = jnp.max(lg_t, axis=1, keepdims=True)
        et = jnp.exp(lg_t - mt)
        wt = (et / jnp.sum(et, axis=1, keepdims=True)).astype(_BF)
        attt_out[i] = _bdot(wt, t16)

        cv2 = jnp.tanh(_bdot(v16, vW_ref[...]) + v_b_ref[...]).astype(_BF)
        c_v = jnp.mean(v_i, axis=0, keepdims=True)
        cv1 = jnp.tanh(cvw_ref[...] * c_v + cvb_ref[...]).astype(_BF)
        lg_v = _bdot_nt(cv1, cv2) * _LAM
        mv = jnp.max(lg_v, axis=1, keepdims=True)
        ev = jnp.exp(lg_v - mv)
        wv = (ev / jnp.sum(ev, axis=1, keepdims=True)).astype(_BF)
        attv_out[i] = _bdot(wv, v16)


def _att_call(t2d, v3, tW16, t_b, vW16, v_b, ctw, ctb, cvw, cvb):
    full = lambda shape: pl.BlockSpec(shape, lambda c, j: tuple(0 for _ in shape))
    out_shapes = (
        jax.ShapeDtypeStruct((_B, _NV, _DH), jnp.float32),
        jax.ShapeDtypeStruct((_B * _L, _DH), jnp.float32),
        jax.ShapeDtypeStruct((_B, 6, _DH), jnp.float32),
        jax.ShapeDtypeStruct((_B, 6, _DH), jnp.float32),
    )
    _ix3 = lambda c, j: (c * (_GB // 2) + j, 0, 0)
    _ix2 = lambda c, j: (c * (_GB // 2) + j, 0)
    out_specs = (
        pl.BlockSpec((_BB, _NV, _DH), _ix3),
        pl.BlockSpec((_BB * _L, _DH), _ix2),
        pl.BlockSpec((_BB, 6, _DH), _ix3),
        pl.BlockSpec((_BB, 6, _DH), _ix3),
    )
    return pl.pallas_call(
        _att_body,
        grid=(2, _GB // 2),
        in_specs=[
            pl.BlockSpec((_BB * _L, _DH), lambda c, j: (c * (_GB // 2) + j, 0)),
            pl.BlockSpec((_BB, _NV, _DH),
                         lambda c, j: (c * (_GB // 2) + j, 0, 0)),
            full((_DH, _DH)), full((1, _DH)),
            full((_DH, _DH)), full((1, _DH)),
            full((6, _DH)), full((6, _DH)), full((6, _DH)), full((6, _DH)),
        ],
        out_specs=out_specs,
        out_shape=out_shapes,
        compiler_params=pltpu.CompilerParams(
            dimension_semantics=("parallel", "arbitrary"),
            vmem_limit_bytes=56 * 1024 * 1024),
        name="oan_attention",
    )(t2d, v3, tW16, t_b, vW16, v_b, ctw, ctb, cvw, cvb)


def kernel(txts, detect, img_W, img_b, conv1_W, conv1_b, conv2_W, conv2_b,
           conv3_W, conv3_b, conv7_W, conv7_b, txt_W, txt_b,
           ct_W, ct_b, t_W, t_b, cv_W, cv_b, v_W, v_b):
    w1 = conv1_W.astype(_BF).transpose(2, 1, 0)
    w2 = conv2_W.astype(_BF).transpose(2, 1, 0)
    w3 = conv3_W.astype(_BF).transpose(2, 1, 0)
    w7 = conv7_W.astype(_BF).transpose(2, 1, 0)
    txt_W16 = txt_W.astype(_BF)
    txtB = jnp.stack([txt_W16[:, j * _DT:(j + 1) * _DT].T for j in range(5)])

    t2d = _txt_call(txts, w1, w2, w3, w7,
                    conv1_b.reshape(1, _DT), conv2_b.reshape(1, _DT),
                    conv3_b.reshape(1, _DT), conv7_b.reshape(1, _DT),
                    txtB, txt_b.reshape(1, _DH))

    v_flat = _img_linear(detect, img_W, img_b.reshape(1, _DH))
    v3 = v_flat.reshape(_B, _NV, _DH)

    ctw = jnp.tile(ct_W[:6, 0:1], (1, _DH))
    ctb = jnp.tile(ct_b[:6, None], (1, _DH))
    cvw = jnp.tile(cv_W[:6, 0:1], (1, _DH))
    cvb = jnp.tile(cv_b[:6, None], (1, _DH))

    vat, tav2d, att_text, att_vis = _att_call(
        t2d, v3, t_W.T.astype(_BF), t_b.reshape(1, _DH),
        v_W.T.astype(_BF), v_b.reshape(1, _DH), ctw, ctb, cvw, cvb)

    t = t2d.reshape(_B, _L, _DH)
    tav = tav2d.reshape(_B, _L, _DH)
    return (v3, t, vat, tav, att_text, att_vis)


# speedup vs baseline: 1.2715x; 1.0675x over previous
import jax
import jax.numpy as jnp
from jax.experimental import pallas as pl
from jax.experimental.pallas import tpu as pltpu

_B, _L, _NV, _DV, _DT, _DH = 128, 80, 36, 12544, 300, 512
_LAM = 9.0
_BF = jnp.bfloat16

_BB1 = 32
_KBLK = 1792
_KT = _DV // _KBLK


def _img_body(x_ref, w_ref, b_ref, o_ref, acc_ref):
    k = pl.program_id(1)

    @pl.when(k == 0)
    def _():
        acc_ref[...] = jnp.zeros_like(acc_ref)

    xb = x_ref[...]
    merged = jnp.concatenate([xb[b] for b in range(_BB1)], axis=0)
    acc_ref[...] += jax.lax.dot_general(
        merged.astype(_BF), w_ref[...].astype(_BF), (((1,), (1,)), ((), ())),
        preferred_element_type=jnp.float32)

    @pl.when(k == _KT - 1)
    def _():
        o_ref[...] = acc_ref[...] + b_ref[...]


def _img_linear(detect, img_W, img_b2):
    return pl.pallas_call(
        _img_body,
        grid=(_B // _BB1, _KT),
        in_specs=[
            pl.BlockSpec((_BB1, _NV, _KBLK), lambda i, k: (i, 0, k)),
            pl.BlockSpec((_DH, _KBLK), lambda i, k: (0, k)),
            pl.BlockSpec((1, _DH), lambda i, k: (0, 0)),
        ],
        out_specs=pl.BlockSpec((_BB1 * _NV, _DH), lambda i, k: (i, 0)),
        out_shape=jax.ShapeDtypeStruct((_B * _NV, _DH), jnp.float32),
        scratch_shapes=[pltpu.VMEM((_BB1 * _NV, _DH), jnp.float32)],
        compiler_params=pltpu.CompilerParams(
            dimension_semantics=("parallel", "arbitrary"),
            vmem_limit_bytes=56 * 1024 * 1024),
        name="img_linear",
    )(detect, img_W, img_b2)


_BB = 8
_GB = _B // _BB


def _shift(x, d):
    if d > 0:
        return jnp.concatenate([x[:, d:, :]] + [x[:, _L - 1:, :]] * d, axis=1)
    if d < 0:
        return jnp.concatenate([x[:, :1, :]] * (-d) + [x[:, : _L + d, :]], axis=1)
    return x


def _bdot(a16, b16):
    return jax.lax.dot_general(a16, b16, (((1,), (0,)), ((), ())),
                               preferred_element_type=jnp.float32)


def _bdot_nt(a16, b16):
    return jax.lax.dot_general(a16, b16, (((1,), (1,)), ((), ())),
                               preferred_element_type=jnp.float32)


def _bdot_tn(a16, b16):
    return jax.lax.dot_general(a16, b16, (((0,), (0,)), ((), ())),
                               preferred_element_type=jnp.float32)


def _txt_body(x_ref, w1_ref, w2_ref, w3_ref, w7_ref,
              b1_ref, b2_ref, b3_ref, b7_ref, txtB_ref, tb_ref, t_out):
    x = x_ref[...].reshape(_BB, _L, _DT)

    def sh(d):
        return _shift(x, d).reshape(_BB * _L, _DT).astype(_BF)

    sm1, s0, s1 = sh(-1), sh(0), sh(1)
    x1 = jnp.tanh(_bdot(s0, w1_ref[0]) + b1_ref[...]).astype(_BF)
    acc = _bdot(x1, txtB_ref[0]) + tb_ref[...]
    s01 = jnp.concatenate([s0, s1], axis=1)
    x2 = jnp.tanh(_bdot(s01, w2_ref[...].reshape(2 * _DT, _DT))
                  + b2_ref[...]).astype(_BF)
    acc = acc + _bdot(x2, txtB_ref[1])
    sm11 = jnp.concatenate([sm1, s01], axis=1)
    x3 = jnp.tanh(_bdot(sm11, w3_ref[...].reshape(3 * _DT, _DT))
                  + b3_ref[...]).astype(_BF)
    acc = acc + _bdot(x3, txtB_ref[2]) + _bdot(x3, txtB_ref[3])
    s7 = jnp.concatenate([sh(-3), sh(-2), sm11, sh(2), sh(3)], axis=1)
    x7 = jnp.tanh(_bdot(s7, w7_ref[...].reshape(7 * _DT, _DT))
                  + b7_ref[...]).astype(_BF)
    acc = acc + _bdot(x7, txtB_ref[4])
    t_out[...] = jnp.tanh(acc)


def _txt_call(txts, w1, w2, w3, w7, b1, b2, b3, b7, txtB, tb):
    full = lambda shape: pl.BlockSpec(shape, lambda c, j: tuple(0 for _ in shape))
    return pl.pallas_call(
        _txt_body,
        grid=(2, _GB // 2),
        in_specs=[
            pl.BlockSpec((_BB * _L, _DT), lambda c, j: (c * (_GB // 2) + j, 0)),
            full((1, _DT, _DT)), full((2, _DT, _DT)), full((3, _DT, _DT)),
            full((7, _DT, _DT)),
            full((1, _DT)), full((1, _DT)), full((1, _DT)), full((1, _DT)),
            full((5, _DT, _DH)), full((1, _DH)),
        ],
        out_specs=pl.BlockSpec((_BB * _L, _DH),
                               lambda c, j: (c * (_GB // 2) + j, 0)),
        out_shape=jax.ShapeDtypeStruct((_B * _L, _DH), jnp.float32),
        compiler_params=pltpu.CompilerParams(
            dimension_semantics=("parallel", "arbitrary"),
            vmem_limit_bytes=56 * 1024 * 1024),
        name="txtnet",
    )(txts, w1, w2, w3, w7, b1, b2, b3, b7, txtB, tb)


def _att_body(t_ref, v_ref, tW_ref, t_b_ref, vW_ref, v_b_ref,
              ctw_ref, ctb_ref, cvw_ref, cvb_ref,
              vat_out, tav_out, attt_out, attv_out):
    tt = t_ref[...]
    tt16 = tt.astype(_BF)
    ct2 = jnp.tanh(_bdot(tt16, tW_ref[...]) + t_b_ref[...]).astype(_BF)

    tn = jnp.sqrt(jnp.sum(tt * tt, axis=1, keepdims=True))
    sims = []
    dens = []
    for i in range(_BB):
        t16 = tt16[i * _L:(i + 1) * _L]
        v_i = v_ref[i]
        v16 = v_i.astype(_BF)
        sims.append(_bdot_nt(t16, v16))
        vn1 = jnp.sqrt(jnp.sum(v_i * v_i, axis=1))
        dens.append(jnp.broadcast_to(vn1[None, :], (_L, _NV)))
    sim = jnp.concatenate(sims, axis=0)
    vnb = jnp.concatenate(dens, axis=0)
    sim_n = sim / jnp.maximum(tn * vnb, 1e-8)

    a = jnp.maximum(sim_n, 0.0)
    ar = a / jnp.maximum(jnp.sum(a, axis=1, keepdims=True), 1e-10)
    er = jnp.exp(ar * _LAM)
    w_v2t = (er / jnp.sum(er, axis=1, keepdims=True)).astype(_BF)

    a3 = a.reshape(_BB, _L, _NV)
    ac = a3 / jnp.maximum(jnp.sum(a3, axis=1, keepdims=True), 1e-10)
    ec = jnp.exp(ac * _LAM)
    w_t2v = (ec / jnp.sum(ec, axis=1, keepdims=True)).reshape(_BB * _L, _NV)
    w_t2v = w_t2v.astype(_BF)

    tsum3 = jnp.sum(tt.reshape(_BB, _L, _DH), axis=1, keepdims=True)

    for i in range(_BB):
        t16 = tt16[i * _L:(i + 1) * _L]
        v_i = v_ref[i]
        v16 = v_i.astype(_BF)
        vat_out[i] = _bdot_tn(w_v2t[i * _L:(i + 1) * _L], t16)
        tav_out[i * _L:(i + 1) * _L] = _bdot(w_t2v[i * _L:(i + 1) * _L], v16)

        c_t = tsum3[i] * (1.0 / _L)
        ct1 = jnp.tanh(ctw_ref[...] * c_t + ctb_ref[...]).astype(_BF)
        lg_t = _bdot_nt(ct1, ct2[i * _L:(i + 1) * _L]) * _LAM
        mt = jnp.max(lg_t, axis=1, keepdims=True)
        et = jnp.exp(lg_t - mt)
        wt = (et / jnp.sum(et, axis=1, keepdims=True)).astype(_BF)
        attt_out[i] = _bdot(wt, t16)

        cv2 = jnp.tanh(_bdot(v16, vW_ref[...]) + v_b_ref[...]).astype(_BF)
        c_v = jnp.mean(v_i, axis=0, keepdims=True)
        cv1 = jnp.tanh(cvw_ref[...] * c_v + cvb_ref[...]).astype(_BF)
        lg_v = _bdot_nt(cv1, cv2) * _LAM
        mv = jnp.max(lg_v, axis=1, keepdims=True)
        ev = jnp.exp(lg_v - mv)
        wv = (ev / jnp.sum(ev, axis=1, keepdims=True)).astype(_BF)
        attv_out[i] = _bdot(wv, v16)


def _att_call(t2d, v3, tW16, t_b, vW16, v_b, ctw, ctb, cvw, cvb):
    full = lambda shape: pl.BlockSpec(shape, lambda c, j: tuple(0 for _ in shape))
    out_shapes = (
        jax.ShapeDtypeStruct((_B, _NV, _DH), jnp.float32),
        jax.ShapeDtypeStruct((_B * _L, _DH), jnp.float32),
        jax.ShapeDtypeStruct((_B, 6, _DH), jnp.float32),
        jax.ShapeDtypeStruct((_B, 6, _DH), jnp.float32),
    )
    _ix3 = lambda c, j: (c * (_GB // 2) + j, 0, 0)
    _ix2 = lambda c, j: (c * (_GB // 2) + j, 0)
    out_specs = (
        pl.BlockSpec((_BB, _NV, _DH), _ix3),
        pl.BlockSpec((_BB * _L, _DH), _ix2),
        pl.BlockSpec((_BB, 6, _DH), _ix3),
        pl.BlockSpec((_BB, 6, _DH), _ix3),
    )
    return pl.pallas_call(
        _att_body,
        grid=(2, _GB // 2),
        in_specs=[
            pl.BlockSpec((_BB * _L, _DH), lambda c, j: (c * (_GB // 2) + j, 0)),
            pl.BlockSpec((_BB, _NV, _DH),
                         lambda c, j: (c * (_GB // 2) + j, 0, 0)),
            full((_DH, _DH)), full((1, _DH)),
            full((_DH, _DH)), full((1, _DH)),
            full((6, _DH)), full((6, _DH)), full((6, _DH)), full((6, _DH)),
        ],
        out_specs=out_specs,
        out_shape=out_shapes,
        compiler_params=pltpu.CompilerParams(
            dimension_semantics=("parallel", "arbitrary"),
            vmem_limit_bytes=56 * 1024 * 1024),
        name="oan_attention",
    )(t2d, v3, tW16, t_b, vW16, v_b, ctw, ctb, cvw, cvb)


def kernel(txts, detect, img_W, img_b, conv1_W, conv1_b, conv2_W, conv2_b,
           conv3_W, conv3_b, conv7_W, conv7_b, txt_W, txt_b,
           ct_W, ct_b, t_W, t_b, cv_W, cv_b, v_W, v_b):
    w1 = conv1_W.astype(_BF).transpose(2, 1, 0)
    w2 = conv2_W.astype(_BF).transpose(2, 1, 0)
    w3 = conv3_W.astype(_BF).transpose(2, 1, 0)
    w7 = conv7_W.astype(_BF).transpose(2, 1, 0)
    txt_W16 = txt_W.astype(_BF)
    txtB = jnp.stack([txt_W16[:, j * _DT:(j + 1) * _DT].T for j in range(5)])

    t2d = _txt_call(txts.reshape(_B * _L, _DT), w1, w2, w3, w7,
                    conv1_b.reshape(1, _DT), conv2_b.reshape(1, _DT),
                    conv3_b.reshape(1, _DT), conv7_b.reshape(1, _DT),
                    txtB, txt_b.reshape(1, _DH))

    v_flat = _img_linear(detect, img_W, img_b.reshape(1, _DH))
    v3 = v_flat.reshape(_B, _NV, _DH)

    ctw = jnp.tile(ct_W[:6, 0:1], (1, _DH))
    ctb = jnp.tile(ct_b[:6, None], (1, _DH))
    cvw = jnp.tile(cv_W[:6, 0:1], (1, _DH))
    cvb = jnp.tile(cv_b[:6, None], (1, _DH))

    vat, tav2d, att_text, att_vis = _att_call(
        t2d, v3, t_W.T.astype(_BF), t_b.reshape(1, _DH),
        v_W.T.astype(_BF), v_b.reshape(1, _DH), ctw, ctb, cvw, cvb)

    t = t2d.reshape(_B, _L, _DH)
    tav = tav2d.reshape(_B, _L, _DH)
    return (v3, t, vat, tav, att_text, att_vis)


# speedup vs baseline: 1.2811x; 1.0076x over previous
import jax
import jax.numpy as jnp
from jax.experimental import pallas as pl
from jax.experimental.pallas import tpu as pltpu

_B, _L, _NV, _DV, _DT, _DH = 128, 80, 36, 12544, 300, 512
_LAM = 9.0
_BF = jnp.bfloat16

_BB1 = 64
_KBLK = 896
_KT = _DV // _KBLK


def _img_body(x_ref, w_ref, b_ref, o_ref, acc_ref):
    k = pl.program_id(1)

    @pl.when(k == 0)
    def _():
        acc_ref[...] = jnp.zeros_like(acc_ref)

    xb = x_ref[...]
    merged = jnp.concatenate([xb[b] for b in range(_BB1)], axis=0)
    acc_ref[...] += jax.lax.dot_general(
        merged.astype(_BF), w_ref[...].astype(_BF), (((1,), (1,)), ((), ())),
        preferred_element_type=jnp.float32)

    @pl.when(k == _KT - 1)
    def _():
        o_ref[...] = acc_ref[...] + b_ref[...]


def _img_linear(detect, img_W, img_b2):
    return pl.pallas_call(
        _img_body,
        grid=(_B // _BB1, _KT),
        in_specs=[
            pl.BlockSpec((_BB1, _NV, _KBLK), lambda i, k: (i, 0, k)),
            pl.BlockSpec((_DH, _KBLK), lambda i, k: (0, k)),
            pl.BlockSpec((1, _DH), lambda i, k: (0, 0)),
        ],
        out_specs=pl.BlockSpec((_BB1 * _NV, _DH), lambda i, k: (i, 0)),
        out_shape=jax.ShapeDtypeStruct((_B * _NV, _DH), jnp.float32),
        scratch_shapes=[pltpu.VMEM((_BB1 * _NV, _DH), jnp.float32)],
        compiler_params=pltpu.CompilerParams(
            dimension_semantics=("parallel", "arbitrary"),
            vmem_limit_bytes=56 * 1024 * 1024),
        name="img_linear",
    )(detect, img_W, img_b2)


_BB = 8
_GB = _B // _BB


def _shift(x, d):
    if d > 0:
        return jnp.concatenate([x[:, d:, :]] + [x[:, _L - 1:, :]] * d, axis=1)
    if d < 0:
        return jnp.concatenate([x[:, :1, :]] * (-d) + [x[:, : _L + d, :]], axis=1)
    return x


def _bdot(a16, b16):
    return jax.lax.dot_general(a16, b16, (((1,), (0,)), ((), ())),
                               preferred_element_type=jnp.float32)


def _bdot_nt(a16, b16):
    return jax.lax.dot_general(a16, b16, (((1,), (1,)), ((), ())),
                               preferred_element_type=jnp.float32)


def _bdot_tn(a16, b16):
    return jax.lax.dot_general(a16, b16, (((0,), (0,)), ((), ())),
                               preferred_element_type=jnp.float32)


def _txt_body(x_ref, w1_ref, w2_ref, w3_ref, w7_ref,
              b1_ref, b2_ref, b3_ref, b7_ref, txtB_ref, tb_ref, t_out):
    x = x_ref[...].reshape(_BB, _L, _DT)

    def sh(d):
        return _shift(x, d).reshape(_BB * _L, _DT).astype(_BF)

    sm1, s0, s1 = sh(-1), sh(0), sh(1)
    x1 = jnp.tanh(_bdot(s0, w1_ref[0]) + b1_ref[...]).astype(_BF)
    acc = _bdot(x1, txtB_ref[0]) + tb_ref[...]
    s01 = jnp.concatenate([s0, s1], axis=1)
    x2 = jnp.tanh(_bdot(s01, w2_ref[...].reshape(2 * _DT, _DT))
                  + b2_ref[...]).astype(_BF)
    acc = acc + _bdot(x2, txtB_ref[1])
    sm11 = jnp.concatenate([sm1, s01], axis=1)
    x3 = jnp.tanh(_bdot(sm11, w3_ref[...].reshape(3 * _DT, _DT))
                  + b3_ref[...]).astype(_BF)
    acc = acc + _bdot(x3, txtB_ref[2]) + _bdot(x3, txtB_ref[3])
    s7 = jnp.concatenate([sh(-3), sh(-2), sm11, sh(2), sh(3)], axis=1)
    x7 = jnp.tanh(_bdot(s7, w7_ref[...].reshape(7 * _DT, _DT))
                  + b7_ref[...]).astype(_BF)
    acc = acc + _bdot(x7, txtB_ref[4])
    t_out[...] = jnp.tanh(acc)


def _txt_call(txts, w1, w2, w3, w7, b1, b2, b3, b7, txtB, tb):
    full = lambda shape: pl.BlockSpec(shape, lambda c, j: tuple(0 for _ in shape))
    return pl.pallas_call(
        _txt_body,
        grid=(2, _GB // 2),
        in_specs=[
            pl.BlockSpec((_BB * _L, _DT), lambda c, j: (c * (_GB // 2) + j, 0)),
            full((1, _DT, _DT)), full((2, _DT, _DT)), full((3, _DT, _DT)),
            full((7, _DT, _DT)),
            full((1, _DT)), full((1, _DT)), full((1, _DT)), full((1, _DT)),
            full((5, _DT, _DH)), full((1, _DH)),
        ],
        out_specs=pl.BlockSpec((_BB * _L, _DH),
                               lambda c, j: (c * (_GB // 2) + j, 0)),
        out_shape=jax.ShapeDtypeStruct((_B * _L, _DH), jnp.float32),
        compiler_params=pltpu.CompilerParams(
            dimension_semantics=("parallel", "arbitrary"),
            vmem_limit_bytes=56 * 1024 * 1024),
        name="txtnet",
    )(txts, w1, w2, w3, w7, b1, b2, b3, b7, txtB, tb)


def _att_body(t_ref, v_ref, tW_ref, t_b_ref, vW_ref, v_b_ref,
              ctw_ref, ctb_ref, cvw_ref, cvb_ref,
              vat_out, tav_out, attt_out, attv_out):
    tt = t_ref[...]
    tt16 = tt.astype(_BF)
    ct2 = jnp.tanh(_bdot(tt16, tW_ref[...]) + t_b_ref[...]).astype(_BF)

    tn = jnp.sqrt(jnp.sum(tt * tt, axis=1, keepdims=True))
    sims = []
    dens = []
    for i in range(_BB):
        t16 = tt16[i * _L:(i + 1) * _L]
        v_i = v_ref[i]
        v16 = v_i.astype(_BF)
        sims.append(_bdot_nt(t16, v16))
        vn1 = jnp.sqrt(jnp.sum(v_i * v_i, axis=1))
        dens.append(jnp.broadcast_to(vn1[None, :], (_L, _NV)))
    sim = jnp.concatenate(sims, axis=0)
    vnb = jnp.concatenate(dens, axis=0)
    sim_n = sim / jnp.maximum(tn * vnb, 1e-8)

    a = jnp.maximum(sim_n, 0.0)
    ar = a / jnp.maximum(jnp.sum(a, axis=1, keepdims=True), 1e-10)
    er = jnp.exp(ar * _LAM)
    w_v2t = (er / jnp.sum(er, axis=1, keepdims=True)).astype(_BF)

    a3 = a.reshape(_BB, _L, _NV)
    ac = a3 / jnp.maximum(jnp.sum(a3, axis=1, keepdims=True), 1e-10)
    ec = jnp.exp(ac * _LAM)
    w_t2v = (ec / jnp.sum(ec, axis=1, keepdims=True)).reshape(_BB * _L, _NV)
    w_t2v = w_t2v.astype(_BF)

    tsum3 = jnp.sum(tt.reshape(_BB, _L, _DH), axis=1, keepdims=True)

    for i in range(_BB):
        t16 = tt16[i * _L:(i + 1) * _L]
        v_i = v_ref[i]
        v16 = v_i.astype(_BF)
        vat_out[i] = _bdot_tn(w_v2t[i * _L:(i + 1) * _L], t16)
        tav_out[i * _L:(i + 1) * _L] = _bdot(w_t2v[i * _L:(i + 1) * _L], v16)

        c_t = tsum3[i] * (1.0 / _L)
        ct1 = jnp.tanh(ctw_ref[...] * c_t + ctb_ref[...]).astype(_BF)
        lg_t = _bdot_nt(ct1, ct2[i * _L:(i + 1) * _L]) * _LAM
        mt = jnp.max(lg_t, axis=1, keepdims=True)
        et = jnp.exp(lg_t - mt)
        wt = (et / jnp.sum(et, axis=1, keepdims=True)).astype(_BF)
        attt_out[i] = _bdot(wt, t16)

        cv2 = jnp.tanh(_bdot(v16, vW_ref[...]) + v_b_ref[...]).astype(_BF)
        c_v = jnp.mean(v_i, axis=0, keepdims=True)
        cv1 = jnp.tanh(cvw_ref[...] * c_v + cvb_ref[...]).astype(_BF)
        lg_v = _bdot_nt(cv1, cv2) * _LAM
        mv = jnp.max(lg_v, axis=1, keepdims=True)
        ev = jnp.exp(lg_v - mv)
        wv = (ev / jnp.sum(ev, axis=1, keepdims=True)).astype(_BF)
        attv_out[i] = _bdot(wv, v16)


def _att_call(t2d, v3, tW16, t_b, vW16, v_b, ctw, ctb, cvw, cvb):
    full = lambda shape: pl.BlockSpec(shape, lambda c, j: tuple(0 for _ in shape))
    out_shapes = (
        jax.ShapeDtypeStruct((_B, _NV, _DH), jnp.float32),
        jax.ShapeDtypeStruct((_B * _L, _DH), jnp.float32),
        jax.ShapeDtypeStruct((_B, 6, _DH), jnp.float32),
        jax.ShapeDtypeStruct((_B, 6, _DH), jnp.float32),
    )
    _ix3 = lambda c, j: (c * (_GB // 2) + j, 0, 0)
    _ix2 = lambda c, j: (c * (_GB // 2) + j, 0)
    out_specs = (
        pl.BlockSpec((_BB, _NV, _DH), _ix3),
        pl.BlockSpec((_BB * _L, _DH), _ix2),
        pl.BlockSpec((_BB, 6, _DH), _ix3),
        pl.BlockSpec((_BB, 6, _DH), _ix3),
    )
    return pl.pallas_call(
        _att_body,
        grid=(2, _GB // 2),
        in_specs=[
            pl.BlockSpec((_BB * _L, _DH), lambda c, j: (c * (_GB // 2) + j, 0)),
            pl.BlockSpec((_BB, _NV, _DH),
                         lambda c, j: (c * (_GB // 2) + j, 0, 0)),
            full((_DH, _DH)), full((1, _DH)),
            full((_DH, _DH)), full((1, _DH)),
            full((6, _DH)), full((6, _DH)), full((6, _DH)), full((6, _DH)),
        ],
        out_specs=out_specs,
        out_shape=out_shapes,
        compiler_params=pltpu.CompilerParams(
            dimension_semantics=("parallel", "arbitrary"),
            vmem_limit_bytes=56 * 1024 * 1024),
        name="oan_attention",
    )(t2d, v3, tW16, t_b, vW16, v_b, ctw, ctb, cvw, cvb)


def kernel(txts, detect, img_W, img_b, conv1_W, conv1_b, conv2_W, conv2_b,
           conv3_W, conv3_b, conv7_W, conv7_b, txt_W, txt_b,
           ct_W, ct_b, t_W, t_b, cv_W, cv_b, v_W, v_b):
    w1 = conv1_W.astype(_BF).transpose(2, 1, 0)
    w2 = conv2_W.astype(_BF).transpose(2, 1, 0)
    w3 = conv3_W.astype(_BF).transpose(2, 1, 0)
    w7 = conv7_W.astype(_BF).transpose(2, 1, 0)
    txt_W16 = txt_W.astype(_BF)
    txtB = jnp.stack([txt_W16[:, j * _DT:(j + 1) * _DT].T for j in range(5)])

    t2d = _txt_call(txts.reshape(_B * _L, _DT), w1, w2, w3, w7,
                    conv1_b.reshape(1, _DT), conv2_b.reshape(1, _DT),
                    conv3_b.reshape(1, _DT), conv7_b.reshape(1, _DT),
                    txtB, txt_b.reshape(1, _DH))

    v_flat = _img_linear(detect, img_W, img_b.reshape(1, _DH))
    v3 = v_flat.reshape(_B, _NV, _DH)

    ctw = jnp.tile(ct_W[:6, 0:1], (1, _DH))
    ctb = jnp.tile(ct_b[:6, None], (1, _DH))
    cvw = jnp.tile(cv_W[:6, 0:1], (1, _DH))
    cvb = jnp.tile(cv_b[:6, None], (1, _DH))

    vat, tav2d, att_text, att_vis = _att_call(
        t2d, v3, t_W.T.astype(_BF), t_b.reshape(1, _DH),
        v_W.T.astype(_BF), v_b.reshape(1, _DH), ctw, ctb, cvw, cvb)

    t = t2d.reshape(_B, _L, _DH)
    tav = tav2d.reshape(_B, _L, _DH)
    return (v3, t, vat, tav, att_text, att_vis)
